# baseline (speedup 1.0000x reference)
import functools

import jax
import jax.numpy as jnp
from jax import lax
from jax.experimental import pallas as pl
from jax.experimental.pallas import tpu as pltpu
from jax.experimental.pallas import tpu_sc as plsc

N = 1_000_000
L = 16
NC = 2
NS = 16
NW = NC * NS
VECS = N // L
BASE_VECS = VECS // NW
REM = VECS - NW * BASE_VECS
CHUNK_VECS = BASE_VECS + 1
BUF = CHUNK_VECS * L

_mesh = plsc.VectorSubcoreMesh(core_axis_name="c", subcore_axis_name="s")


@functools.partial(
    pl.kernel,
    mesh=_mesh,
    out_type=jax.ShapeDtypeStruct((NC, 2 * L), jnp.float32),
    scratch_types=[
        pltpu.VMEM((BUF,), jnp.float32),
        pltpu.VMEM((2 * L,), jnp.float32),
        pltpu.VMEM((NS * 2 * L,), jnp.float32),
        pltpu.VMEM_SHARED((NS * 2 * L,), jnp.float32),
    ],
)
def _margin_partials(score_hbm, out_hbm, buf, stage, allbuf, shared):
    c = lax.axis_index("c")
    s = lax.axis_index("s")
    wid = s * NC + c
    base_vec = wid * BASE_VECS + jnp.minimum(wid, REM)
    base = base_vec * L

    pltpu.sync_copy(
        score_hbm.at[pl.ds(base, BASE_VECS * L)], buf.at[pl.ds(0, BASE_VECS * L)]
    )
    buf[pl.ds(BASE_VECS * L, L)] = jnp.zeros((L,), jnp.float32)

    @pl.when(wid < REM)
    def _():
        pltpu.sync_copy(
            score_hbm.at[pl.ds(base + BASE_VECS * L, L)],
            buf.at[pl.ds(BASE_VECS * L, L)],
        )

    def body(i, carry):
        vs, vc = carry
        v = buf[pl.ds(i * L, L)]
        vs = vs + jnp.minimum(v, 0.0)
        vc = vc + jnp.where(v < 0.0, 1.0, 0.0)
        return vs, vc

    zero = jnp.zeros((L,), jnp.float32)
    vs, vc = lax.fori_loop(0, CHUNK_VECS, body, (zero, zero))

    stage[pl.ds(0, L)] = vs
    stage[pl.ds(L, L)] = vc
    pltpu.sync_copy(stage, shared.at[pl.ds(s * 2 * L, 2 * L)])
    plsc.subcore_barrier()

    @pl.when(s == 0)
    def _():
        pltpu.sync_copy(shared, allbuf)
        acc_s = jnp.zeros((L,), jnp.float32)
        acc_c = jnp.zeros((L,), jnp.float32)
        for r in range(NS):
            acc_s = acc_s + allbuf[pl.ds(r * 2 * L, L)]
            acc_c = acc_c + allbuf[pl.ds(r * 2 * L + L, L)]
        stage[pl.ds(0, L)] = acc_s
        stage[pl.ds(L, L)] = acc_c
        pltpu.sync_copy(stage, out_hbm.at[c])


def kernel(preds, lables, score):
    out = _margin_partials(score)
    total = jnp.sum(out[:, :L])
    count = jnp.sum(out[:, L:])
    return total / count


# speedup vs baseline: 1.1525x; 1.1525x over previous
import functools

import jax
import jax.numpy as jnp
from jax import lax
from jax.experimental import pallas as pl
from jax.experimental.pallas import tpu as pltpu
from jax.experimental.pallas import tpu_sc as plsc

N = 1_000_000
L = 16
NC = 2
NS = 16
NW = NC * NS
VECS = N // L
BASE_VECS = VECS // NW
REM = VECS - NW * BASE_VECS
U = 4
PAD_VECS = 1960
HALF = 980
BUF = PAD_VECS * L

_mesh = plsc.VectorSubcoreMesh(core_axis_name="c", subcore_axis_name="s")


@functools.partial(
    pl.kernel,
    mesh=_mesh,
    out_type=jax.ShapeDtypeStruct((NC, 2 * L), jnp.float32),
    scratch_types=[
        pltpu.VMEM((BUF,), jnp.float32),
        pltpu.VMEM((2 * L,), jnp.float32),
        pltpu.VMEM((NS * 2 * L,), jnp.float32),
        pltpu.VMEM_SHARED((NS * 2 * L,), jnp.float32),
        pltpu.SemaphoreType.DMA,
        pltpu.SemaphoreType.DMA,
    ],
)
def _margin_partials(score_hbm, out_hbm, buf, stage, allbuf, shared, sem0, sem1):
    c = lax.axis_index("c")
    s = lax.axis_index("s")
    wid = s * NC + c
    base_vec = wid * BASE_VECS + jnp.minimum(wid, REM)
    base = base_vec * L

    cp0 = pltpu.async_copy(
        score_hbm.at[pl.ds(base, HALF * L)], buf.at[pl.ds(0, HALF * L)], sem0
    )
    cp1 = pltpu.async_copy(
        score_hbm.at[pl.ds(base + HALF * L, (BASE_VECS - HALF) * L)],
        buf.at[pl.ds(HALF * L, (BASE_VECS - HALF) * L)],
        sem1,
    )
    zf = jnp.zeros((L,), jnp.float32)
    for pad_vec in range(BASE_VECS + 1, PAD_VECS):
        buf[pl.ds(pad_vec * L, L)] = zf

    @pl.when(wid >= REM)
    def _():
        buf[pl.ds(BASE_VECS * L, L)] = zf

    def piece(lo, hi, carry):
        @plsc.parallel_loop(lo, hi, step=U, unroll=2, carry=carry)
        def body(i, accs):
            vss = list(accs[:U])
            vcs = list(accs[U:])
            for u in range(U):
                v = buf[pl.ds((i + u) * L, L)]
                vss[u] = vss[u] + jnp.minimum(v, 0.0)
                vcs[u] = vcs[u] + jnp.where(v < 0.0, 1.0, 0.0)
            return (*vss, *vcs)

        return body

    cp0.wait()
    accs = piece(0, HALF, (zf,) * (2 * U))
    cp1.wait()

    @pl.when(wid < REM)
    def _():
        pltpu.sync_copy(
            score_hbm.at[pl.ds(base + BASE_VECS * L, L)],
            buf.at[pl.ds(BASE_VECS * L, L)],
        )

    accs = piece(HALF, PAD_VECS, accs)
    vs = (accs[0] + accs[1]) + (accs[2] + accs[3])
    vc = (accs[4] + accs[5]) + (accs[6] + accs[7])

    stage[pl.ds(0, L)] = vs
    stage[pl.ds(L, L)] = vc
    pltpu.sync_copy(stage, shared.at[pl.ds(s * 2 * L, 2 * L)])
    plsc.subcore_barrier()

    @pl.when(s == 0)
    def _():
        pltpu.sync_copy(shared, allbuf)
        acc_s = jnp.zeros((L,), jnp.float32)
        acc_c = jnp.zeros((L,), jnp.float32)
        for r in range(NS):
            acc_s = acc_s + allbuf[pl.ds(r * 2 * L, L)]
            acc_c = acc_c + allbuf[pl.ds(r * 2 * L + L, L)]
        stage[pl.ds(0, L)] = acc_s
        stage[pl.ds(L, L)] = acc_c
        pltpu.sync_copy(stage, out_hbm.at[c])


def kernel(preds, lables, score):
    out = _margin_partials(score)
    total = jnp.sum(out[:, :L])
    count = jnp.sum(out[:, L:])
    return total / count


# speedup vs baseline: 1.1668x; 1.0124x over previous
import functools

import jax
import jax.numpy as jnp
from jax import lax
from jax.experimental import pallas as pl
from jax.experimental.pallas import tpu as pltpu
from jax.experimental.pallas import tpu_sc as plsc

N = 1_000_000
L = 16
NC = 2
NS = 16
NW = NC * NS
VECS = N // L
BASE_VECS = VECS // NW
REM = VECS - NW * BASE_VECS
U = 4
PAD_VECS = 1968
HALF = 976
BUF = PAD_VECS * L

_mesh = plsc.VectorSubcoreMesh(core_axis_name="c", subcore_axis_name="s")


@functools.partial(
    pl.kernel,
    mesh=_mesh,
    out_type=jax.ShapeDtypeStruct((NW, 2 * L), jnp.float32),
    scratch_types=[
        pltpu.VMEM((BUF,), jnp.float32),
        pltpu.VMEM((2 * L,), jnp.float32),
        pltpu.SemaphoreType.DMA,
        pltpu.SemaphoreType.DMA,
        pltpu.SemaphoreType.DMA,
    ],
)
def _margin_partials(score_hbm, out_hbm, buf, stage, sem0, sem1, sem2):
    c = lax.axis_index("c")
    s = lax.axis_index("s")
    wid = s * NC + c
    base_vec = wid * BASE_VECS + jnp.minimum(wid, REM)
    base = base_vec * L

    cp0 = pltpu.async_copy(
        score_hbm.at[pl.ds(base, HALF * L)], buf.at[pl.ds(0, HALF * L)], sem0
    )
    cp1 = pltpu.async_copy(
        score_hbm.at[pl.ds(base + HALF * L, (BASE_VECS - HALF) * L)],
        buf.at[pl.ds(HALF * L, (BASE_VECS - HALF) * L)],
        sem1,
    )

    extra_src = score_hbm.at[pl.ds(base + BASE_VECS * L, L)]
    extra_dst = buf.at[pl.ds(BASE_VECS * L, L)]

    @pl.when(wid < REM)
    def _():
        pltpu.async_copy(extra_src, extra_dst, sem2)

    zf = jnp.zeros((L,), jnp.float32)
    for pad_vec in range(BASE_VECS + 1, PAD_VECS):
        buf[pl.ds(pad_vec * L, L)] = zf

    @pl.when(wid >= REM)
    def _():
        buf[pl.ds(BASE_VECS * L, L)] = zf

    def piece(lo, hi, carry):
        @plsc.parallel_loop(lo, hi, step=U, unroll=4, carry=carry)
        def body(i, accs):
            vss = list(accs[:U])
            vcs = list(accs[U:])
            for u in range(U):
                v = buf[pl.ds((i + u) * L, L)]
                vss[u] = vss[u] + jnp.minimum(v, 0.0)
                vcs[u] = vcs[u] + jnp.where(v < 0.0, 1.0, 0.0)
            return (*vss, *vcs)

        return body

    cp0.wait()
    accs = piece(0, HALF, (zf,) * (2 * U))
    cp1.wait()

    @pl.when(wid < REM)
    def _():
        pltpu.make_async_copy(extra_src, extra_dst, sem2).wait()

    accs = piece(HALF, PAD_VECS, accs)
    vs = (accs[0] + accs[1]) + (accs[2] + accs[3])
    vc = (accs[4] + accs[5]) + (accs[6] + accs[7])

    stage[pl.ds(0, L)] = vs
    stage[pl.ds(L, L)] = vc
    pltpu.sync_copy(stage, out_hbm.at[wid])


def kernel(preds, lables, score):
    out = _margin_partials(score)
    total = jnp.sum(out[:, :L])
    count = jnp.sum(out[:, L:])
    return total / count
